# baseline (speedup 1.0000x reference)
import jax
import jax.numpy as jnp
import numpy as np
from jax import lax
from jax.experimental import pallas as pl
from jax.experimental.pallas import tpu as pltpu
from jax.experimental.pallas import tpu_sc as plsc

N = 10000
E = 320000
H = 128
ED = 16
G = 64

NC = 2
NS = 16
NW = NC * NS
EPW = E // NW
CH = 80
NCHUNK = 126
EPWP = NCHUNK * CH
EP = NW * EPWP
RPS = 632
NPAD = NS * RPS


def _precompute_body(ns_ref, w1mn_ref, gs_ref, w1mg_ref, b1m_ref,
                     w1ug_ref, b1u_ref, p_ref, cm_ref, cu_ref):
    p_ref[...] = jnp.dot(ns_ref[...], w1mn_ref[...],
                         preferred_element_type=jnp.float32)
    g = gs_ref[...]
    cm_ref[...] = jnp.dot(g, w1mg_ref[...],
                          preferred_element_type=jnp.float32) + b1m_ref[...]
    cu_ref[...] = jnp.dot(g, w1ug_ref[...],
                          preferred_element_type=jnp.float32) + b1u_ref[...]


def _precompute(ns, w1mn, gs, w1mg, b1m, w1ug, b1u):
    return pl.pallas_call(
        _precompute_body,
        out_shape=[
            jax.ShapeDtypeStruct((N, H), jnp.float32),
            jax.ShapeDtypeStruct((1, H), jnp.float32),
            jax.ShapeDtypeStruct((1, H), jnp.float32),
        ],
    )(ns, w1mn, gs, w1mg, b1m, w1ug, b1u)


_QBLK = 3360


def _q_body(ese_ref, eso_ref, w1me_ref, cm_ref, q_ref):
    w = w1me_ref[...]
    cm = cm_ref[...]
    qa = (jnp.dot(ese_ref[...], w, preferred_element_type=jnp.float32)
          + cm).astype(jnp.bfloat16)
    qb = (jnp.dot(eso_ref[...], w, preferred_element_type=jnp.float32)
          + cm).astype(jnp.bfloat16)
    pa = jax.lax.bitcast_convert_type(qa, jnp.uint16).astype(jnp.uint32)
    pb = jax.lax.bitcast_convert_type(qb, jnp.uint16).astype(jnp.uint32)
    q_ref[...] = jax.lax.bitcast_convert_type(pa | (pb << 16), jnp.int32)


def _q_kernel(ese, eso, w1me, cm):
    grid = (EP // _QBLK,)
    return pl.pallas_call(
        _q_body,
        grid=grid,
        in_specs=[
            pl.BlockSpec((_QBLK // 2, ED), lambda i: (i, 0)),
            pl.BlockSpec((_QBLK // 2, ED), lambda i: (i, 0)),
            pl.BlockSpec((ED, H), lambda i: (0, 0)),
            pl.BlockSpec((1, H), lambda i: (0, 0)),
        ],
        out_specs=pl.BlockSpec((_QBLK // 2, H), lambda i: (i, 0)),
        out_shape=jax.ShapeDtypeStruct((EP // 2, H), jnp.int32),
    )(ese, eso, w1me, cm)


def _sc_body(p_hbm, q_hbm, src_hbm, dst_hbm,
             s_out, deg_out,
             src0, src1, src2, dst0, dst1, dst2, hb0, hb1, hb2,
             qb0, qb1, qb2, ones_v, dbuf, s_sh, deg_sh,
             semq, semsi, semdi, semg, sems, semd):
    c = lax.axis_index("c")
    s = lax.axis_index("s")
    wid = s * NC + c
    row0 = pl.multiple_of(s * RPS, 8)
    SV = (src0, src1, src2)
    DV = (dst0, dst1, dst2)
    HB = (hb0, hb1, hb2)
    QB = (qb0, qb1, qb2)

    def zero_row(r, cc):
        for cb in range(H // 16):
            hb0[r, pl.ds(cb * 16, 16)] = jnp.zeros((16,), jnp.float32)
        return cc

    lax.fori_loop(0, CH, zero_row, 0)
    for k in range(640 // 16):
        dbuf[pl.ds(k * 16, 16)] = jnp.zeros((16,), jnp.float32)
    for k in range(128 // 16):
        ones_v[pl.ds(k * 16, 16)] = jnp.ones((16,), jnp.float32)
    off = 0
    for ln in (CH,) * (RPS // CH) + (RPS % CH,):
        pltpu.sync_copy(hb0.at[pl.ds(0, ln)],
                        s_sh.at[pl.ds(row0 + off, ln)])
        off += ln
    pltpu.sync_copy(dbuf.at[pl.ds(0, RPS)], deg_sh.at[pl.ds(row0, RPS)])
    plsc.subcore_barrier()

    def q_start(j, b):
        base = pl.multiple_of(wid * EPWP + j * CH, 16)
        base2 = pl.multiple_of(wid * (EPWP // 2) + j * (CH // 2), 8)
        pltpu.async_copy(q_hbm.at[pl.ds(base2, CH // 2)], QB[b], semq.at[b])
        pltpu.async_copy(src_hbm.at[pl.ds(base, CH)], SV[b], semsi.at[b])
        pltpu.async_copy(dst_hbm.at[pl.ds(base, CH)], DV[b], semdi.at[b])

    def si_wait(b):
        pltpu.make_async_copy(src_hbm.at[pl.ds(0, CH)], SV[b],
                              semsi.at[b]).wait()

    def q_wait(b):
        pltpu.make_async_copy(q_hbm.at[pl.ds(0, CH // 2)], QB[b],
                              semq.at[b]).wait()
        pltpu.make_async_copy(dst_hbm.at[pl.ds(0, CH)], DV[b],
                              semdi.at[b]).wait()

    def g_start(b):
        pltpu.async_copy(p_hbm.at[SV[b]], HB[b], semg.at[b])

    def g_wait(b):
        pltpu.make_async_copy(p_hbm.at[SV[b]], HB[b], semg.at[b]).wait()

    def combine(b):
        hbuf = HB[b]
        qbuf = QB[b]

        mask = jnp.int32(-65536)

        def rows(rr, cc):
            r0 = 2 * rr
            r1 = 2 * rr + 1
            for g in range(H // 16):
                sl = pl.ds(g * 16, 16)
                qv = qbuf[rr, sl]
                qa = jax.lax.bitcast_convert_type(jnp.left_shift(qv, 16),
                                                  jnp.float32)
                qb = jax.lax.bitcast_convert_type(jnp.bitwise_and(qv, mask),
                                                  jnp.float32)
                hbuf[r0, sl] = jnp.maximum(hbuf[r0, sl] + qa, 0.0)
                hbuf[r1, sl] = jnp.maximum(hbuf[r1, sl] + qb, 0.0)
            return cc

        lax.fori_loop(0, CH // 2, rows, 0)

    def s_start(b):
        pltpu.async_copy(HB[b], s_sh.at[DV[b]], sems.at[b], add=True)
        pltpu.async_copy(ones_v.at[pl.ds(0, CH)], deg_sh.at[DV[b]],
                         semd.at[b], add=True)

    def s_wait(b):
        pltpu.make_async_copy(HB[b], s_sh.at[DV[b]], sems.at[b]).wait()
        pltpu.make_async_copy(ones_v.at[pl.ds(0, CH)], deg_sh.at[DV[b]],
                              semd.at[b]).wait()

    def step(j, r, g_prefetch=True, q_prefetch=True, wait_m=True):
        p, n, m = r, (r + 1) % 3, (r + 2) % 3
        if g_prefetch:
            si_wait(n)
            g_start(n)
        g_wait(p)
        q_wait(p)
        combine(p)
        s_start(p)
        if wait_m:
            s_wait(m)
        if q_prefetch:
            q_start(j + 2, m)

    q_start(0, 0)
    si_wait(0)
    g_start(0)
    q_start(1, 1)
    step(0, 0, wait_m=False)
    step(1, 1)
    step(2, 2)

    def triple(i, cc):
        j0 = i * 3
        step(j0, 0)
        step(j0 + 1, 1)
        step(j0 + 2, 2)
        return cc

    lax.fori_loop(1, NCHUNK // 3 - 1, triple, 0)
    step(NCHUNK - 3, 0)
    step(NCHUNK - 2, 1, q_prefetch=False)
    step(NCHUNK - 1, 2, g_prefetch=False, q_prefetch=False)
    s_wait(2)
    plsc.subcore_barrier()

    off = 0
    for ln in (CH,) * (RPS // CH) + (RPS % CH,):
        pltpu.sync_copy(s_sh.at[pl.ds(row0 + off, ln)],
                        hb0.at[pl.ds(0, ln)])
        pltpu.sync_copy(hb0.at[pl.ds(0, ln)],
                        s_out.at[c, pl.ds(row0 + off, ln)])
        off += ln
    pltpu.sync_copy(deg_sh.at[pl.ds(row0, RPS)], dbuf.at[pl.ds(0, RPS)])
    dout0 = pl.multiple_of(c * NPAD + row0, 8)
    pltpu.sync_copy(dbuf.at[pl.ds(0, RPS)], deg_out.at[pl.ds(dout0, RPS)])


def _sc_scatter(p, q, src, dst):
    mesh = plsc.VectorSubcoreMesh(core_axis_name="c", subcore_axis_name="s")
    fn = pl.kernel(
        _sc_body,
        out_type=[
            jax.ShapeDtypeStruct((NC, NPAD, H), jnp.float32),
            jax.ShapeDtypeStruct((NC * NPAD,), jnp.float32),
        ],
        mesh=mesh,
        scratch_types=(
            [pltpu.VMEM((CH,), jnp.int32)] * 6
            + [pltpu.VMEM((CH, H), jnp.float32)] * 3
            + [pltpu.VMEM((CH // 2, H), jnp.int32)] * 3
            + [pltpu.VMEM((128,), jnp.float32),
               pltpu.VMEM((640,), jnp.float32),
               pltpu.VMEM_SHARED((NPAD, H), jnp.float32),
               pltpu.VMEM_SHARED((NPAD,), jnp.float32)]
            + [pltpu.SemaphoreType.DMA((3,))] * 6
        ),
    )
    return fn(p, q, src, dst)


_BBLK = 2000


def _final_body(s_ref, deg_ref, ns_ref, w2m_ref,
                b2m_ref, w1un_ref, w1ua_ref, cu_ref, w2u_ref, b2u_ref,
                gamma_ref, beta_ref, out_ref):
    s_sum = s_ref[0] + s_ref[1]
    deg = deg_ref[0, 0] + deg_ref[1, 0]
    aggsum = jnp.dot(s_sum, w2m_ref[...],
                     preferred_element_type=jnp.float32) + deg * b2m_ref[...]
    agg = aggsum / jnp.maximum(deg, 1.0)
    ns = ns_ref[...]
    u = jnp.dot(ns, w1un_ref[...], preferred_element_type=jnp.float32)
    u += jnp.dot(agg, w1ua_ref[...], preferred_element_type=jnp.float32)
    u = jnp.maximum(u + cu_ref[...], 0.0)
    delta = jnp.dot(u, w2u_ref[...],
                    preferred_element_type=jnp.float32) + b2u_ref[...]
    x = ns + delta
    mu = jnp.mean(x, axis=-1, keepdims=True)
    xc = x - mu
    var = jnp.mean(xc * xc, axis=-1, keepdims=True)
    out_ref[...] = xc * lax.rsqrt(var + 1e-5) * gamma_ref[...] + beta_ref[...]


def _finalize(s_parts, degr, ns, w2m, b2m, w1un, w1ua, cu, w2u, b2u,
              gamma, beta):
    grid = (N // _BBLK,)
    full = lambda shape: pl.BlockSpec(shape, lambda i: tuple(0 for _ in shape))
    return pl.pallas_call(
        _final_body,
        grid=grid,
        in_specs=[
            pl.BlockSpec((NC, _BBLK, H), lambda i: (0, i, 0)),
            pl.BlockSpec((NC, 1, _BBLK, 1), lambda i: (0, i, 0, 0)),
            pl.BlockSpec((_BBLK, H), lambda i: (i, 0)),
            full((H, H)),
            full((1, H)),
            full((H, H)),
            full((H, H)),
            full((1, H)),
            full((H, H)),
            full((1, H)),
            full((1, H)),
            full((1, H)),
        ],
        out_specs=pl.BlockSpec((_BBLK, H), lambda i: (i, 0)),
        out_shape=jax.ShapeDtypeStruct((N, H), jnp.float32),
    )(s_parts, degr, ns, w2m, b2m, w1un, w1ua, cu, w2u, b2u, gamma, beta)


def kernel(node_state, edge_index, edge_state, global_state,
           W1m, b1m, W2m, b2m, W1u, b1u, W2u, b2u, gamma, beta):
    src = edge_index[0]
    dst = edge_index[1]
    w1mn, w1me, w1mg = W1m[:H], W1m[H:H + ED], W1m[H + ED:]
    w1un, w1ua, w1ug = W1u[:H], W1u[H:2 * H], W1u[2 * H:]
    gs = global_state.reshape(1, G)
    b1ur = b1u.reshape(1, H)

    b1mr = b1m.reshape(1, H)

    pad = ((0, 0), (0, EPWP - EPW))
    src_p = jnp.pad(src.reshape(NW, EPW), pad).reshape(EP)
    dst_p = jnp.pad(dst.reshape(NW, EPW), pad,
                    constant_values=NPAD - 1).reshape(EP)
    es_p = jnp.pad(edge_state.reshape(NW, EPW, ED),
                   (*pad, (0, 0))).reshape(EP, ED)

    p, cm, cu = _precompute(node_state, w1mn, gs, w1mg, b1mr, w1ug, b1ur)
    q = _q_kernel(es_p[0::2], es_p[1::2], w1me, cm)
    s_parts, deg_parts = _sc_scatter(p, q, src_p, dst_p)
    degr = deg_parts.reshape(NC, NPAD)[:, :N].reshape(NC, N // _BBLK, _BBLK, 1)
    return _finalize(s_parts, degr, node_state, W2m, b2m.reshape(1, H),
                     w1un, w1ua, cu, W2u, b2u.reshape(1, H),
                     gamma.reshape(1, H), beta.reshape(1, H))


# speedup vs baseline: 2.7509x; 2.7509x over previous
import jax
import jax.numpy as jnp
import numpy as np
from jax import lax
from jax.experimental import pallas as pl
from jax.experimental.pallas import tpu as pltpu
from jax.experimental.pallas import tpu_sc as plsc

N = 10000
E = 320000
H = 128
ED = 16
G = 64

NC = 2
NS = 16
NW = NC * NS
EPW = E // NW
CH = 80
NCHUNK = EPW // CH
RPS = 632
NPAD = NS * RPS


def _precompute_body(ns_ref, w1mn_ref, gs_ref, w1mg_ref, b1m_ref,
                     w1ug_ref, b1u_ref, p_ref, cm_ref, cu_ref):
    p_ref[...] = jnp.dot(ns_ref[...], w1mn_ref[...],
                         preferred_element_type=jnp.float32)
    g = gs_ref[...]
    cm_ref[...] = jnp.dot(g, w1mg_ref[...],
                          preferred_element_type=jnp.float32) + b1m_ref[...]
    cu_ref[...] = jnp.dot(g, w1ug_ref[...],
                          preferred_element_type=jnp.float32) + b1u_ref[...]


def _precompute(ns, w1mn, gs, w1mg, b1m, w1ug, b1u):
    return pl.pallas_call(
        _precompute_body,
        out_shape=[
            jax.ShapeDtypeStruct((N, H), jnp.float32),
            jax.ShapeDtypeStruct((1, H), jnp.float32),
            jax.ShapeDtypeStruct((1, H), jnp.float32),
        ],
    )(ns, w1mn, gs, w1mg, b1m, w1ug, b1u)


_QBLK = 1600


def _q_body(es_ref, w1me_ref, cm_ref, q_ref):
    q_ref[...] = jnp.dot(es_ref[...], w1me_ref[...],
                         preferred_element_type=jnp.float32) + cm_ref[...]


def _q_kernel(es, w1me, cm):
    grid = (E // _QBLK,)
    return pl.pallas_call(
        _q_body,
        grid=grid,
        in_specs=[
            pl.BlockSpec((_QBLK, ED), lambda i: (i, 0)),
            pl.BlockSpec((ED, H), lambda i: (0, 0)),
            pl.BlockSpec((1, H), lambda i: (0, 0)),
        ],
        out_specs=pl.BlockSpec((_QBLK, H), lambda i: (i, 0)),
        out_shape=jax.ShapeDtypeStruct((E, H), jnp.float32),
    )(es, w1me, cm)


def _sc_body(p_hbm, q_hbm, src_hbm, dst_hbm,
             s_out, deg_out,
             src0, src1, src2, dst0, dst1, dst2, hb0, hb1, hb2,
             ones_v, dbuf, s_sh, deg_sh,
             semq, semsi, semdi, semg, sems, semd):
    c = lax.axis_index("c")
    s = lax.axis_index("s")
    wid = s * NC + c
    row0 = pl.multiple_of(s * RPS, 8)
    SV = (src0, src1, src2)
    DV = (dst0, dst1, dst2)
    HB = (hb0, hb1, hb2)

    def zero_row(r, cc):
        for cb in range(H // 16):
            hb0[r, pl.ds(cb * 16, 16)] = jnp.zeros((16,), jnp.float32)
        return cc

    lax.fori_loop(0, CH, zero_row, 0)
    for k in range(640 // 16):
        dbuf[pl.ds(k * 16, 16)] = jnp.zeros((16,), jnp.float32)
    for k in range(128 // 16):
        ones_v[pl.ds(k * 16, 16)] = jnp.ones((16,), jnp.float32)
    off = 0
    for ln in (CH,) * (RPS // CH) + (RPS % CH,):
        pltpu.sync_copy(hb0.at[pl.ds(0, ln)],
                        s_sh.at[pl.ds(row0 + off, ln)])
        off += ln
    pltpu.sync_copy(dbuf.at[pl.ds(0, RPS)], deg_sh.at[pl.ds(row0, RPS)])
    plsc.subcore_barrier()

    def q_start(j, b):
        base = pl.multiple_of(wid * EPW + j * CH, 16)
        pltpu.async_copy(q_hbm.at[pl.ds(base, CH)], HB[b], semq.at[b])
        pltpu.async_copy(src_hbm.at[pl.ds(base, CH)], SV[b], semsi.at[b])
        pltpu.async_copy(dst_hbm.at[pl.ds(base, CH)], DV[b], semdi.at[b])

    def q_wait(b):
        pltpu.make_async_copy(q_hbm.at[pl.ds(0, CH)], HB[b],
                              semq.at[b]).wait()
        pltpu.make_async_copy(src_hbm.at[pl.ds(0, CH)], SV[b],
                              semsi.at[b]).wait()
        pltpu.make_async_copy(dst_hbm.at[pl.ds(0, CH)], DV[b],
                              semdi.at[b]).wait()

    def g_start(b):
        pltpu.async_copy(p_hbm.at[SV[b]], HB[b], semg.at[b], add=True)

    def g_wait(b):
        pltpu.make_async_copy(p_hbm.at[SV[b]], HB[b], semg.at[b]).wait()

    def combine(b):
        buf = HB[b]

        @plsc.parallel_loop(0, CH, step=1, unroll=4)
        def _(r):
            for cb in range(H // 16):
                sl = pl.ds(cb * 16, 16)
                buf[r, sl] = jnp.maximum(buf[r, sl], 0.0)

    def s_start(b):
        pltpu.async_copy(HB[b], s_sh.at[DV[b]], sems.at[b], add=True)
        pltpu.async_copy(ones_v.at[pl.ds(0, CH)], deg_sh.at[DV[b]],
                         semd.at[b], add=True)

    def s_wait(b):
        pltpu.make_async_copy(HB[b], s_sh.at[DV[b]], sems.at[b]).wait()
        pltpu.make_async_copy(ones_v.at[pl.ds(0, CH)], deg_sh.at[DV[b]],
                              semd.at[b]).wait()

    def step(j, r, g_prefetch=True, q_prefetch=True, wait_m=True):
        p, n, m = r, (r + 1) % 3, (r + 2) % 3
        if g_prefetch:
            q_wait(n)
            g_start(n)
        g_wait(p)
        combine(p)
        s_start(p)
        if wait_m:
            s_wait(m)
        if q_prefetch:
            q_start(j + 2, m)

    q_start(0, 0)
    q_wait(0)
    g_start(0)
    q_start(1, 1)
    step(0, 0, wait_m=False)
    step(1, 1)
    step(2, 2)

    def triple(i, cc):
        j0 = i * 3
        step(j0, 0)
        step(j0 + 1, 1)
        step(j0 + 2, 2)
        return cc

    lax.fori_loop(1, (NCHUNK - 2) // 3, triple, 0)
    step(NCHUNK - 2, 0, q_prefetch=False)
    step(NCHUNK - 1, 1, g_prefetch=False, q_prefetch=False)
    s_wait(1)
    plsc.subcore_barrier()

    off = 0
    for ln in (CH,) * (RPS // CH) + (RPS % CH,):
        pltpu.sync_copy(s_sh.at[pl.ds(row0 + off, ln)],
                        hb0.at[pl.ds(0, ln)])
        pltpu.sync_copy(hb0.at[pl.ds(0, ln)],
                        s_out.at[c, pl.ds(row0 + off, ln)])
        off += ln
    pltpu.sync_copy(deg_sh.at[pl.ds(row0, RPS)], dbuf.at[pl.ds(0, RPS)])
    dout0 = pl.multiple_of(c * NPAD + row0, 8)
    pltpu.sync_copy(dbuf.at[pl.ds(0, RPS)], deg_out.at[pl.ds(dout0, RPS)])


def _sc_scatter(p, q, src, dst):
    mesh = plsc.VectorSubcoreMesh(core_axis_name="c", subcore_axis_name="s")
    fn = pl.kernel(
        _sc_body,
        out_type=[
            jax.ShapeDtypeStruct((NC, NPAD, H), jnp.float32),
            jax.ShapeDtypeStruct((NC * NPAD,), jnp.float32),
        ],
        mesh=mesh,
        scratch_types=(
            [pltpu.VMEM((CH,), jnp.int32)] * 6
            + [pltpu.VMEM((CH, H), jnp.float32)] * 3
            + [pltpu.VMEM((128,), jnp.float32),
               pltpu.VMEM((640,), jnp.float32),
               pltpu.VMEM_SHARED((NPAD, H), jnp.float32),
               pltpu.VMEM_SHARED((NPAD,), jnp.float32)]
            + [pltpu.SemaphoreType.DMA((3,))] * 6
        ),
    )
    return fn(p, q, src, dst)


_BBLK = 2000


def _final_body(s_ref, deg_ref, ns_ref, w2m_ref,
                b2m_ref, w1un_ref, w1ua_ref, cu_ref, w2u_ref, b2u_ref,
                gamma_ref, beta_ref, out_ref):
    s_sum = s_ref[0] + s_ref[1]
    deg = deg_ref[0, 0] + deg_ref[1, 0]
    aggsum = jnp.dot(s_sum, w2m_ref[...],
                     preferred_element_type=jnp.float32) + deg * b2m_ref[...]
    agg = aggsum / jnp.maximum(deg, 1.0)
    ns = ns_ref[...]
    u = jnp.dot(ns, w1un_ref[...], preferred_element_type=jnp.float32)
    u += jnp.dot(agg, w1ua_ref[...], preferred_element_type=jnp.float32)
    u = jnp.maximum(u + cu_ref[...], 0.0)
    delta = jnp.dot(u, w2u_ref[...],
                    preferred_element_type=jnp.float32) + b2u_ref[...]
    x = ns + delta
    mu = jnp.mean(x, axis=-1, keepdims=True)
    xc = x - mu
    var = jnp.mean(xc * xc, axis=-1, keepdims=True)
    out_ref[...] = xc * lax.rsqrt(var + 1e-5) * gamma_ref[...] + beta_ref[...]


def _finalize(s_parts, degr, ns, w2m, b2m, w1un, w1ua, cu, w2u, b2u,
              gamma, beta):
    grid = (N // _BBLK,)
    full = lambda shape: pl.BlockSpec(shape, lambda i: tuple(0 for _ in shape))
    return pl.pallas_call(
        _final_body,
        grid=grid,
        in_specs=[
            pl.BlockSpec((NC, _BBLK, H), lambda i: (0, i, 0)),
            pl.BlockSpec((NC, 1, _BBLK, 1), lambda i: (0, i, 0, 0)),
            pl.BlockSpec((_BBLK, H), lambda i: (i, 0)),
            full((H, H)),
            full((1, H)),
            full((H, H)),
            full((H, H)),
            full((1, H)),
            full((H, H)),
            full((1, H)),
            full((1, H)),
            full((1, H)),
        ],
        out_specs=pl.BlockSpec((_BBLK, H), lambda i: (i, 0)),
        out_shape=jax.ShapeDtypeStruct((N, H), jnp.float32),
    )(s_parts, degr, ns, w2m, b2m, w1un, w1ua, cu, w2u, b2u, gamma, beta)


def kernel(node_state, edge_index, edge_state, global_state,
           W1m, b1m, W2m, b2m, W1u, b1u, W2u, b2u, gamma, beta):
    src = edge_index[0]
    dst = edge_index[1]
    w1mn, w1me, w1mg = W1m[:H], W1m[H:H + ED], W1m[H + ED:]
    w1un, w1ua, w1ug = W1u[:H], W1u[H:2 * H], W1u[2 * H:]
    gs = global_state.reshape(1, G)
    b1ur = b1u.reshape(1, H)

    b1mr = b1m.reshape(1, H)

    p, cm, cu = _precompute(node_state, w1mn, gs, w1mg, b1mr, w1ug, b1ur)
    q = _q_kernel(edge_state, w1me, cm)
    s_parts, deg_parts = _sc_scatter(p, q, src, dst)
    degr = deg_parts.reshape(NC, NPAD)[:, :N].reshape(NC, N // _BBLK, _BBLK, 1)
    return _finalize(s_parts, degr, node_state, W2m, b2m.reshape(1, H),
                     w1un, w1ua, cu, W2u, b2u.reshape(1, H),
                     gamma.reshape(1, H), beta.reshape(1, H))
